# baseline (speedup 1.0000x reference)
import jax
import jax.numpy as jnp
from jax import lax
from jax.experimental import pallas as pl
from jax.experimental.pallas import tpu as pltpu

N = 1000000
CI = 262144
NI = 4
CO = 524288
NO = 2
AI = CI // 128
AO = CO // 128
BIG = 2**30


def _v2(v, a):
    return pltpu.einshape("(ab)->ab", v, a=a, b=128,
                          assert_is_tile_preserving=True)


def _v1(v):
    return pltpu.einshape("ab->(ab)", v, assert_is_tile_preserving=True)


def _lin(a, base):
    r = lax.broadcasted_iota(jnp.int32, (a, 128), 0)
    c = lax.broadcasted_iota(jnp.int32, (a, 128), 1)
    return r * 128 + c + base


def _body(x_ref, out_ref, idx_ref, max_ref, lm_ref):
    i = pl.program_id(0)

    @pl.when(i < NI - 1)
    def _plain_max():
        lm_ref[0] = jnp.max(_v2(x_ref[...], AI))

    @pl.when(i == NI - 1)
    def _masked_max():
        xv = _v2(x_ref[...], AI)
        lm_ref[0] = jnp.max(jnp.where(_lin(AI, i * CI) < N, xv, -jnp.inf))

    @pl.when(i < NI)
    def _phase1():
        lm = lm_ref[0]

        @pl.when((i == 0) | (lm > max_ref[0]))
        def _new_max():
            xv = _v2(x_ref[...], AI)
            cand = jnp.where(xv == lm, _lin(AI, i * CI), BIG)
            max_ref[0] = lm
            idx_ref[0] = jnp.min(cand)

    @pl.when(i >= NI)
    def _phase2():
        j = i - NI
        out_ref[...] = _v1((_lin(AO, j * CO) == idx_ref[0]).astype(jnp.int32))


def kernel(x):
    return pl.pallas_call(
        _body,
        grid=(NI + NO,),
        in_specs=[pl.BlockSpec((CI,), lambda i: (jnp.minimum(i, NI - 1),))],
        out_specs=pl.BlockSpec((CO,), lambda i: (jnp.maximum(i - NI, 0),)),
        out_shape=jax.ShapeDtypeStruct((N,), jnp.int32),
        scratch_shapes=[
            pltpu.SMEM((1,), jnp.int32),
            pltpu.SMEM((1,), jnp.float32),
            pltpu.SMEM((1,), jnp.float32),
        ],
    )(x)


# speedup vs baseline: 1.0721x; 1.0721x over previous
import jax
import jax.numpy as jnp
from jax import lax
from jax.experimental import pallas as pl
from jax.experimental.pallas import tpu as pltpu

N = 1000000
CHB = 524288
NB = 2
A = CHB // 128
BIG = 2**30


def _view2d(v):
    return pltpu.einshape("(ab)->ab", v, a=A, b=128,
                          assert_is_tile_preserving=True)


def _view1d(v):
    return pltpu.einshape("ab->(ab)", v, assert_is_tile_preserving=True)


def _lin(i):
    r = lax.broadcasted_iota(jnp.int32, (A, 128), 0)
    c = lax.broadcasted_iota(jnp.int32, (A, 128), 1)
    return r * 128 + c + i * CHB


def _body(x_ref, out_ref, idx_ref, max_ref, lm_ref):
    i = pl.program_id(0)

    @pl.when(i < NB - 1)
    def _plain_max():
        lm_ref[0] = jnp.max(_view2d(x_ref[...]))

    @pl.when(i == NB - 1)
    def _masked_max():
        xv = _view2d(x_ref[...])
        lm_ref[0] = jnp.max(jnp.where(_lin(i) < N, xv, -jnp.inf))

    @pl.when(i < NB)
    def _phase1():
        lm = lm_ref[0]

        @pl.when((i == 0) | (lm > max_ref[0]))
        def _new_max():
            xv = _view2d(x_ref[...])
            cand = jnp.where(xv == lm, _lin(i), BIG)
            max_ref[0] = lm
            idx_ref[0] = jnp.min(cand)

    @pl.when(i >= NB)
    def _phase2():
        j = i - NB
        out_ref[...] = _view1d((_lin(j) == idx_ref[0]).astype(jnp.int32))


def kernel(x):
    return pl.pallas_call(
        _body,
        grid=(2 * NB,),
        in_specs=[pl.BlockSpec((CHB,), lambda i: (jnp.minimum(i, NB - 1),))],
        out_specs=pl.BlockSpec((CHB,), lambda i: (jnp.maximum(i - NB, 0),)),
        out_shape=jax.ShapeDtypeStruct((N,), jnp.int32),
        scratch_shapes=[
            pltpu.SMEM((1,), jnp.int32),
            pltpu.SMEM((1,), jnp.float32),
            pltpu.SMEM((1,), jnp.float32),
        ],
    )(x)
